# baseline (speedup 1.0000x reference)
import functools

import jax
import jax.numpy as jnp
from jax import lax
from jax.experimental import pallas as pl
from jax.experimental.pallas import tpu as pltpu
from jax.experimental.pallas import tpu_sc as plsc

N = 512
NC2 = N * (N - 1) // 2
B = 128

SC_CORES = 2
SC_SUBCORES = 16
NW = SC_CORES * SC_SUBCORES
BATCH_PER_W = B // NW

RB = 32
NBLK = N // RB

def _tri(i):
    return (i * (i - 1)) // 2

_A = []
_L = []
for _k in range(NBLK):
    a = (_tri(RB * _k) // 8) * 8
    end = _tri(RB * (_k + 1))
    l = -(-(end - a) // 8) * 8
    _A.append(a)
    _L.append(l)
CHUNK_MAX = max(_L) + N + 16


def _sc_build_body(flat_hbm, p_hbm, chunk_v, strip_v):
    wid = lax.axis_index("s") * SC_CORES + lax.axis_index("c")
    lane = lax.iota(jnp.int32, 16)

    def per_batch(bb, carry):
        b = wid * BATCH_PER_W + bb
        for k in range(NBLK):
            src_off = pl.multiple_of(b * NC2 + _A[k], 8)
            pltpu.sync_copy(flat_hbm.at[pl.ds(src_off, _L[k])],
                            chunk_v.at[pl.ds(0, _L[k])])
            w_k = RB * (k + 1)

            def per_row(r, c2, k=k, w_k=w_k):
                i = RB * k + r
                off = (i * (i - 1)) // 2 - _A[k]
                for g in range(w_k // 16):
                    idx = off + g * 16 + lane
                    v = plsc.load_gather(chunk_v, [idx])
                    strip_v[r, pl.ds(g * 16, 16)] = v
                return c2

            lax.fori_loop(0, RB, per_row, 0)
            pltpu.sync_copy(strip_v, p_hbm.at[b, pl.ds(RB * k, RB)])
        return carry

    lax.fori_loop(0, BATCH_PER_W, per_batch, 0)


@functools.cache
def _sc_build():
    return pl.kernel(
        _sc_build_body,
        mesh=plsc.VectorSubcoreMesh(core_axis_name="c", subcore_axis_name="s"),
        out_type=jax.ShapeDtypeStruct((B, N, N), jnp.float32),
        scratch_types=[
            pltpu.VMEM((CHUNK_MAX,), jnp.float32),
            pltpu.VMEM((RB, N), jnp.float32),
        ],
        compiler_params=pltpu.CompilerParams(needs_layout_passes=False),
    )


STRIP = 128
NSTRIP = N // STRIP


def _sym_body(r_ref, o_ref):
    bi = pl.program_id(1)
    bj = pl.program_id(2)
    ig = jax.lax.broadcasted_iota(jnp.int32, (STRIP, STRIP), 0) + bi * STRIP
    jg = jax.lax.broadcasted_iota(jnp.int32, (STRIP, STRIP), 1) + bj * STRIP
    r = r_ref[0]
    o_ref[0] = jnp.where(jg < ig, r, 0.0) + jnp.where(ig < jg, r.T, 0.0)


def _sym_call(p, interpret=False):
    b = p.shape[0]
    return pl.pallas_call(
        _sym_body,
        grid=(b, NSTRIP, NSTRIP),
        in_specs=[
            pl.BlockSpec(
                (1, STRIP, STRIP),
                lambda i, bi, bj: (i, jnp.maximum(bi, bj), jnp.minimum(bi, bj)),
            ),
        ],
        out_specs=pl.BlockSpec((1, STRIP, STRIP), lambda i, bi, bj: (i, bi, bj)),
        out_shape=jax.ShapeDtypeStruct((b, N, N), jnp.float32),
        interpret=interpret,
    )(p)


def kernel(decompFE):
    p = _sc_build()(decompFE.reshape(-1))
    return _sym_call(p)


# speedup vs baseline: 3.1273x; 3.1273x over previous
import functools

import jax
import jax.numpy as jnp
from jax import lax
from jax.experimental import pallas as pl
from jax.experimental.pallas import tpu as pltpu
from jax.experimental.pallas import tpu_sc as plsc

N = 512
NC2 = N * (N - 1) // 2
B = 128

SC_CORES = 2
SC_SUBCORES = 16
NW = SC_CORES * SC_SUBCORES
BATCH_PER_W = B // NW

RB = 32
NBLK = N // RB

def _tri(i):
    return (i * (i - 1)) // 2

_A = []
_L = []
for _k in range(NBLK):
    a = (_tri(RB * _k) // 8) * 8
    end = _tri(RB * (_k + 1))
    l = -(-(end - a) // 8) * 8
    _A.append(a)
    _L.append(l)
CHUNK_MAX = max(_L) + N + 16


def _sc_build_body(flat_hbm, p_hbm, chunk_v, strip_v):
    wid = lax.axis_index("s") * SC_CORES + lax.axis_index("c")
    lane = lax.iota(jnp.int32, 16)

    def per_batch(bb, carry):
        b = wid * BATCH_PER_W + bb
        for k in range(NBLK):
            src_off = pl.multiple_of(b * NC2 + _A[k], 8)
            pltpu.sync_copy(flat_hbm.at[pl.ds(src_off, _L[k])],
                            chunk_v.at[pl.ds(0, _L[k])])
            w_k = RB * (k + 1)

            def per_row(r, c2, k=k, w_k=w_k):
                i = RB * k + r
                off = (i * (i - 1)) // 2 - _A[k]
                for g in range(w_k // 16):
                    idx = off + g * 16 + lane
                    v = plsc.load_gather(chunk_v, [idx])
                    strip_v[r, pl.ds(g * 16, 16)] = v
                return c2

            lax.fori_loop(0, RB, per_row, 0)
            pltpu.sync_copy(strip_v, p_hbm.at[b, pl.ds(RB * k, RB)])
        return carry

    lax.fori_loop(0, BATCH_PER_W, per_batch, 0)


@functools.cache
def _sc_build():
    return pl.kernel(
        _sc_build_body,
        mesh=plsc.VectorSubcoreMesh(core_axis_name="c", subcore_axis_name="s"),
        out_type=jax.ShapeDtypeStruct((B, N, N), jnp.float32),
        scratch_types=[
            pltpu.VMEM((CHUNK_MAX,), jnp.float32),
            pltpu.VMEM((RB, N), jnp.float32),
        ],
        compiler_params=pltpu.CompilerParams(needs_layout_passes=False),
    )


STRIP = 128
NSTRIP = N // STRIP


BB = 16


def _sym_body(r_ref, o_ref):
    bi = pl.program_id(1)
    bj = pl.program_id(2)
    ig = jax.lax.broadcasted_iota(jnp.int32, (STRIP, STRIP), 0) + bi * STRIP
    jg = jax.lax.broadcasted_iota(jnp.int32, (STRIP, STRIP), 1) + bj * STRIP
    r = r_ref[...]
    rt = jnp.swapaxes(r, 1, 2)
    o_ref[...] = (jnp.where((jg < ig)[None], r, 0.0)
                  + jnp.where((ig < jg)[None], rt, 0.0))


def _sym_call(p, interpret=False):
    b = p.shape[0]
    return pl.pallas_call(
        _sym_body,
        grid=(b // BB, NSTRIP, NSTRIP),
        in_specs=[
            pl.BlockSpec(
                (BB, STRIP, STRIP),
                lambda i, bi, bj: (i, jnp.maximum(bi, bj), jnp.minimum(bi, bj)),
            ),
        ],
        out_specs=pl.BlockSpec((BB, STRIP, STRIP), lambda i, bi, bj: (i, bi, bj)),
        out_shape=jax.ShapeDtypeStruct((b, N, N), jnp.float32),
        interpret=interpret,
    )(p)


def kernel(decompFE):
    p = _sc_build()(decompFE.reshape(-1))
    return _sym_call(p)


# speedup vs baseline: 4.0392x; 1.2916x over previous
import functools

import jax
import jax.numpy as jnp
from jax import lax
from jax.experimental import pallas as pl
from jax.experimental.pallas import tpu as pltpu
from jax.experimental.pallas import tpu_sc as plsc

N = 512
NC2 = N * (N - 1) // 2
B = 128

SC_CORES = 2
SC_SUBCORES = 16
NW = SC_CORES * SC_SUBCORES
BATCH_PER_W = B // NW

RB = 32
NBLK = N // RB

def _tri(i):
    return (i * (i - 1)) // 2

_A = []
_L = []
for _k in range(NBLK):
    a = (_tri(RB * _k) // 8) * 8
    end = _tri(RB * (_k + 1))
    l = -(-(end - a) // 8) * 8
    _A.append(a)
    _L.append(l)
CHUNK_MAX = max(_L) + N + 16

_W128 = [min(N, -(-(RB * (_k + 1)) // 128) * 128) for _k in range(NBLK)]


def _sc_build_body(flat_hbm, p_hbm, chunk0, chunk1, strip0, strip1,
                   cs0, cs1, ss0, ss1):
    wid = lax.axis_index("s") * SC_CORES + lax.axis_index("c")
    lane = lax.iota(jnp.int32, 16)
    chunks = [chunk0, chunk1]
    strips = [strip0, strip1]
    csem = [cs0, cs1]
    ssem = [ss0, ss1]

    def per_batch(bb, carry):
        b = wid * BATCH_PER_W + bb

        def chunk_start(k, slot):
            src_off = pl.multiple_of(b * NC2 + _A[k], 8)
            return pltpu.async_copy(flat_hbm.at[pl.ds(src_off, _L[k])],
                                    chunks[slot].at[pl.ds(0, _L[k])],
                                    csem[slot])

        h = chunk_start(0, 0)
        pending = [None, None]
        for k in range(NBLK):
            cur = k & 1
            hc = h
            if k + 1 < NBLK:
                h = chunk_start(k + 1, 1 - cur)
            hc.wait()
            if pending[cur] is not None:
                pending[cur].wait()
            w_k = RB * (k + 1)

            def per_row(r, c2, k=k, w_k=w_k, cur=cur):
                i = RB * k + r
                off = (i * (i - 1)) // 2 - _A[k]
                for g in range(w_k // 16):
                    idx = off + g * 16 + lane
                    v = plsc.load_gather(chunks[cur], [idx])
                    strips[cur][r, pl.ds(g * 16, 16)] = v
                return c2

            lax.fori_loop(0, RB, per_row, 0)
            pending[cur] = pltpu.async_copy(
                strips[cur].at[pl.ds(0, RB), pl.ds(0, _W128[k])],
                p_hbm.at[b, pl.ds(RB * k, RB), pl.ds(0, _W128[k])],
                ssem[cur])
        for ps in pending:
            if ps is not None:
                ps.wait()
        return carry

    lax.fori_loop(0, BATCH_PER_W, per_batch, 0)


@functools.cache
def _sc_build():
    return pl.kernel(
        _sc_build_body,
        mesh=plsc.VectorSubcoreMesh(core_axis_name="c", subcore_axis_name="s"),
        out_type=jax.ShapeDtypeStruct((B, N, N), jnp.float32),
        scratch_types=[
            pltpu.VMEM((CHUNK_MAX,), jnp.float32),
            pltpu.VMEM((CHUNK_MAX,), jnp.float32),
            pltpu.VMEM((RB, N), jnp.float32),
            pltpu.VMEM((RB, N), jnp.float32),
            pltpu.SemaphoreType.DMA,
            pltpu.SemaphoreType.DMA,
            pltpu.SemaphoreType.DMA,
            pltpu.SemaphoreType.DMA,
        ],
        compiler_params=pltpu.CompilerParams(needs_layout_passes=False),
    )


STRIP = 128
NSTRIP = N // STRIP


BB = 16


def _sym_body(r_ref, o_ref):
    bi = pl.program_id(1)
    bj = pl.program_id(2)
    ig = jax.lax.broadcasted_iota(jnp.int32, (STRIP, STRIP), 0) + bi * STRIP
    jg = jax.lax.broadcasted_iota(jnp.int32, (STRIP, STRIP), 1) + bj * STRIP
    r = r_ref[...]
    rt = jnp.swapaxes(r, 1, 2)
    o_ref[...] = (jnp.where((jg < ig)[None], r, 0.0)
                  + jnp.where((ig < jg)[None], rt, 0.0))


def _sym_call(p, interpret=False):
    b = p.shape[0]
    return pl.pallas_call(
        _sym_body,
        grid=(b // BB, NSTRIP, NSTRIP),
        in_specs=[
            pl.BlockSpec(
                (BB, STRIP, STRIP),
                lambda i, bi, bj: (i, jnp.maximum(bi, bj), jnp.minimum(bi, bj)),
            ),
        ],
        out_specs=pl.BlockSpec((BB, STRIP, STRIP), lambda i, bi, bj: (i, bi, bj)),
        out_shape=jax.ShapeDtypeStruct((b, N, N), jnp.float32),
        interpret=interpret,
    )(p)


def kernel(decompFE):
    p = _sc_build()(decompFE.reshape(-1))
    return _sym_call(p)
